# baseline (speedup 1.0000x reference)
import functools

import jax
import jax.numpy as jnp
from jax import lax
from jax.experimental import pallas as pl
from jax.experimental.pallas import tpu as pltpu
from jax.experimental.pallas import tpu_sc as plsc

_N = 10000
_E = 160000
_F = 16
_G = 8
_NC = 2
_NS = 16
_NW = _NC * _NS
_CHUNK = 125
_NROWS = _E // _CHUNK
_CPW = _NROWS // _NW
_EPW = _CPW * _CHUNK
_NPT = _N // _NS
_EBG = 1000


def _vmesh():
    return plsc.VectorSubcoreMesh(core_axis_name="c", subcore_axis_name="s")


def _sc_params():
    return pltpu.CompilerParams(use_tc_tiling_on_sc=False)


def _full_spec(a):
    nd = a.ndim
    return pl.BlockSpec(a.shape, lambda i, _nd=nd: (0,) * _nd)


def _tc_embed(xg, w1, b1, w2, b2, w3, b3):
    def body(x_ref, w1r, b1r, w2r, b2r, w3r, b3r, o_ref):
        h = jnp.maximum(
            jnp.dot(x_ref[...], w1r[...], preferred_element_type=jnp.float32)
            + b1r[...], 0.0)
        h = jnp.maximum(
            jnp.dot(h, w2r[...], preferred_element_type=jnp.float32)
            + b2r[...], 0.0)
        o_ref[...] = (
            jnp.dot(h, w3r[...], preferred_element_type=jnp.float32)
            + b3r[...])

    ng, dg = xg.shape
    return pl.pallas_call(
        body,
        grid=(1,),
        in_specs=[pl.BlockSpec((ng, dg), lambda i: (0, 0)),
                  _full_spec(w1), _full_spec(b1), _full_spec(w2),
                  _full_spec(b2), _full_spec(w3), _full_spec(b3)],
        out_specs=pl.BlockSpec((ng, _G * _F), lambda i: (0, 0)),
        out_shape=jax.ShapeDtypeStruct((ng, _G * _F), jnp.float32),
    )(xg, w1, b1, w2, b2, w3, b3)


def _tc_msg(eag, hsg, w1, b1, w2, b2, w3, b3, rmat, cmat):
    def body(ea_ref, hs_ref, w1r, b1r, w2r, b2r, w3r, bgr, rr, cr, o_ref):
        bf = jnp.bfloat16
        hsb = hs_ref[...].astype(bf)
        t = jnp.maximum(
            jnp.dot(ea_ref[...], w1r[...], preferred_element_type=jnp.float32)
            + b1r[...], 0.0).astype(bf)
        t = jnp.maximum(
            jnp.dot(t, w2r[...], preferred_element_type=jnp.float32)
            + b2r[...], 0.0).astype(bf)
        p = jnp.dot(t, w3r[...],
                    preferred_element_type=jnp.float32).astype(bf)
        hrep = jnp.dot(hsb, rr[...],
                       preferred_element_type=jnp.float32).astype(bf)
        o_ref[...] = (
            jnp.dot(hrep * p, cr[...], preferred_element_type=jnp.float32)
            + jnp.dot(hsb, bgr[...], preferred_element_type=jnp.float32))

    eg = _E // _G
    return pl.pallas_call(
        body,
        grid=(eg // _EBG,),
        in_specs=[pl.BlockSpec((_EBG, _G * _F), lambda i: (i, 0)),
                  pl.BlockSpec((_EBG, _G * _F), lambda i: (i, 0)),
                  _full_spec(w1), _full_spec(b1), _full_spec(w2),
                  _full_spec(b2), _full_spec(w3), _full_spec(b3),
                  _full_spec(rmat), _full_spec(cmat)],
        out_specs=pl.BlockSpec((_EBG, _G * _F), lambda i: (i, 0)),
        out_shape=jax.ShapeDtypeStruct((eg, _G * _F), jnp.float32),
    )(eag, hsg, w1, b1, w2, b2, w3, b3, rmat, cmat)


def _tc_finalize(agg2g, cnt2g, hg, rootg, biasg):
    def body(a_ref, c_ref, h_ref, rr, br, o_ref):
        a = a_ref[0] + a_ref[1]
        cnt = c_ref[0] + c_ref[1]
        mean = a / jnp.maximum(cnt, 1.0)
        o_ref[...] = jnp.maximum(
            mean
            + jnp.dot(h_ref[...], rr[...], preferred_element_type=jnp.float32)
            + br[...], 0.0)

    ng = _N // _G
    return pl.pallas_call(
        body,
        grid=(1,),
        in_specs=[pl.BlockSpec((_NC, ng, _G * _F), lambda i: (0, 0, 0)),
                  pl.BlockSpec((_NC, ng, _G * _F), lambda i: (0, 0, 0)),
                  pl.BlockSpec((ng, _G * _F), lambda i: (0, 0)),
                  _full_spec(rootg), _full_spec(biasg)],
        out_specs=pl.BlockSpec((ng, _G * _F), lambda i: (0, 0)),
        out_shape=jax.ShapeDtypeStruct((ng, _G * _F), jnp.float32),
    )(agg2g, cnt2g, hg, rootg, biasg)


def _sc_gather(h, src2):
    @functools.partial(
        pl.kernel,
        out_type=jax.ShapeDtypeStruct((_E, _F), jnp.float32),
        mesh=_vmesh(),
        scratch_types=[pltpu.VMEM((_CPW, _CHUNK), jnp.int32),
                       pltpu.VMEM((_EPW, _F), jnp.float32),
                       pltpu.SemaphoreType.DMA,
                       pltpu.SemaphoreType.DMA],
        compiler_params=_sc_params(),
    )
    def k(h_hbm, s_hbm, o_hbm, idx_v, rows_v, sem0, sem1):
        wid = lax.axis_index("s") * _NC + lax.axis_index("c")
        r0 = wid * _CPW
        e0 = wid * _EPW
        pltpu.async_copy(s_hbm.at[pl.ds(r0, _CPW)], idx_v, sem0).wait()

        @pl.loop(0, _CPW)
        def _fire(j):
            pltpu.async_copy(h_hbm.at[idx_v.at[j]],
                             rows_v.at[pl.ds(j * _CHUNK, _CHUNK)], sem1)

        pltpu.make_async_copy(o_hbm.at[pl.ds(e0, _EPW)], rows_v, sem1).wait()
        pltpu.async_copy(rows_v, o_hbm.at[pl.ds(e0, _EPW)], sem0).wait()

    return k(h, src2)


def _sc_scatter(msg, dst2, zeros):
    @functools.partial(
        pl.kernel,
        out_type=jax.ShapeDtypeStruct((_NC, _N, _F), jnp.float32),
        mesh=_vmesh(),
        scratch_types=[pltpu.VMEM((_CPW, _CHUNK), jnp.int32),
                       pltpu.VMEM((_EPW, _F), jnp.float32),
                       pltpu.VMEM_SHARED((_N, _F), jnp.float32),
                       pltpu.SemaphoreType.DMA,
                       pltpu.SemaphoreType.DMA],
        compiler_params=_sc_params(),
    )
    def k(m_hbm, d_hbm, z_hbm, o_hbm, idx_v, msg_v, acc_sh, sem0, sem1):
        cid = lax.axis_index("c")
        sid = lax.axis_index("s")
        wid = sid * _NC + cid
        r0 = wid * _CPW
        e0 = wid * _EPW
        n0 = sid * _NPT
        cp_z = pltpu.async_copy(z_hbm.at[pl.ds(n0, _NPT)],
                                acc_sh.at[pl.ds(n0, _NPT)], sem0)
        cp_i = pltpu.async_copy(d_hbm.at[pl.ds(r0, _CPW)], idx_v, sem1)
        cp_m = pltpu.async_copy(m_hbm.at[pl.ds(e0, _EPW)], msg_v, sem1)
        cp_z.wait()
        cp_i.wait()
        cp_m.wait()
        plsc.subcore_barrier()

        @pl.loop(0, _CPW)
        def _fire(j):
            pltpu.async_copy(msg_v.at[pl.ds(j * _CHUNK, _CHUNK)],
                             acc_sh.at[idx_v.at[j]], sem0, add=True)

        pltpu.make_async_copy(m_hbm.at[pl.ds(e0, _EPW)], msg_v, sem0).wait()
        plsc.subcore_barrier()
        pltpu.sync_copy(acc_sh.at[pl.ds(n0, _NPT)],
                        o_hbm.at[cid, pl.ds(n0, _NPT)])

    return k(msg, dst2, zeros)


def _sc_count(dst2, zeros):
    @functools.partial(
        pl.kernel,
        out_type=jax.ShapeDtypeStruct((_NC, _N, _F), jnp.float32),
        mesh=_vmesh(),
        scratch_types=[pltpu.VMEM((_CPW, _CHUNK), jnp.int32),
                       pltpu.VMEM((_CHUNK, _F), jnp.float32),
                       pltpu.VMEM_SHARED((_N, _F), jnp.float32),
                       pltpu.SemaphoreType.DMA],
        compiler_params=_sc_params(),
    )
    def k(d_hbm, z_hbm, o_hbm, idx_v, ones_v, acc_sh, sem0):
        cid = lax.axis_index("c")
        sid = lax.axis_index("s")
        wid = sid * _NC + cid
        r0 = wid * _CPW
        n0 = sid * _NPT
        cp_z = pltpu.async_copy(z_hbm.at[pl.ds(n0, _NPT)],
                                acc_sh.at[pl.ds(n0, _NPT)], sem0)
        cp_i = pltpu.async_copy(d_hbm.at[pl.ds(r0, _CPW)], idx_v, sem0)

        @pl.loop(0, _CHUNK)
        def _fill(i):
            ones_v[i, :] = jnp.ones((_F,), jnp.float32)

        cp_z.wait()
        cp_i.wait()
        plsc.subcore_barrier()

        @pl.loop(0, _CPW)
        def _fire(j):
            pltpu.sync_copy(ones_v, acc_sh.at[idx_v.at[j]], add=True)

        plsc.subcore_barrier()
        pltpu.sync_copy(acc_sh.at[pl.ds(n0, _NPT)],
                        o_hbm.at[cid, pl.ds(n0, _NPT)])

    return k(dst2, zeros)


def kernel(x, edge_index, edge_attr, params):
    src2 = edge_index[0].reshape(_NROWS, _CHUNK)
    dst2 = edge_index[1].reshape(_NROWS, _CHUNK)
    zeros = jnp.zeros((_N, _F), jnp.float32)
    eye8 = jnp.eye(_G, dtype=jnp.float32)
    kron8 = lambda w: jnp.kron(eye8, w)
    tile8 = lambda b: jnp.tile(b, (_G,)).reshape(1, -1)

    rmat = (jnp.arange(_F * _F)[None, :] // _F
            == jnp.arange(_F)[:, None]).astype(jnp.float32)
    cmat = (jnp.arange(_F * _F)[:, None] % _F
            == jnp.arange(_F)[None, :]).astype(jnp.float32)

    e = params['emb']
    xg = x.reshape(_N // _G, -1)
    hg = _tc_embed(xg, kron8(e['W1']), tile8(e['b1']),
                   kron8(e['W2']), tile8(e['b2']),
                   kron8(e['W3']), tile8(e['b3']))
    cnt2 = _sc_count(dst2, zeros)
    cnt2g = cnt2.reshape(_NC, _N // _G, _G * _F)

    bfc = lambda a: a.astype(jnp.bfloat16)
    eag = bfc(edge_attr.reshape(_E // _G, _G * _F))
    for c in params['convs']:
        hs = _sc_gather(hg.reshape(_N, _F), src2)
        bmat = c['eb3'].reshape(_F, _F)
        msgg = _tc_msg(eag, hs.reshape(_E // _G, _G * _F),
                       bfc(kron8(c['eW1'])), tile8(c['eb1']),
                       bfc(kron8(c['eW2'])), tile8(c['eb2']),
                       bfc(kron8(c['eW3'])), bfc(kron8(bmat)),
                       bfc(kron8(rmat)), bfc(kron8(cmat)))
        agg2 = _sc_scatter(msgg.reshape(_E, _F), dst2, zeros)
        hg = _tc_finalize(agg2.reshape(_NC, _N // _G, _G * _F), cnt2g,
                          hg, kron8(c['root']), tile8(c['bias']))
    return hg.reshape(_N, _F)


# speedup vs baseline: 1.1225x; 1.1225x over previous
import functools

import jax
import jax.numpy as jnp
from jax import lax
from jax.experimental import pallas as pl
from jax.experimental.pallas import tpu as pltpu
from jax.experimental.pallas import tpu_sc as plsc

_N = 10000
_E = 160000
_F = 16
_G = 8
_NC = 2
_NS = 16
_NW = _NC * _NS
_CHUNK = 125
_NROWS = _E // _CHUNK
_CPW = _NROWS // _NW
_EPW = _CPW * _CHUNK
_NPT = _N // _NS
_EBG = 2000


def _vmesh():
    return plsc.VectorSubcoreMesh(core_axis_name="c", subcore_axis_name="s")


def _sc_params():
    return pltpu.CompilerParams(use_tc_tiling_on_sc=False)


def _full_spec(a):
    nd = a.ndim
    return pl.BlockSpec(a.shape, lambda i, _nd=nd: (0,) * _nd)


def _tc_embed(xg, w1, b1, w2, b2, w3, b3):
    def body(x_ref, w1r, b1r, w2r, b2r, w3r, b3r, o_ref):
        h = jnp.maximum(
            jnp.dot(x_ref[...], w1r[...], preferred_element_type=jnp.float32)
            + b1r[...], 0.0)
        h = jnp.maximum(
            jnp.dot(h, w2r[...], preferred_element_type=jnp.float32)
            + b2r[...], 0.0)
        o_ref[...] = (
            jnp.dot(h, w3r[...], preferred_element_type=jnp.float32)
            + b3r[...])

    ng, dg = xg.shape
    return pl.pallas_call(
        body,
        grid=(1,),
        in_specs=[pl.BlockSpec((ng, dg), lambda i: (0, 0)),
                  _full_spec(w1), _full_spec(b1), _full_spec(w2),
                  _full_spec(b2), _full_spec(w3), _full_spec(b3)],
        out_specs=pl.BlockSpec((ng, _G * _F), lambda i: (0, 0)),
        out_shape=jax.ShapeDtypeStruct((ng, _G * _F), jnp.float32),
    )(xg, w1, b1, w2, b2, w3, b3)


def _tc_msg(eag, hsg, w1, b1, w2, b2, w3, b3, rmat, cmat):
    def body(ea_ref, hs_ref, w1r, b1r, w2r, b2r, w3r, bgr, rr, cr, o_ref):
        hs = hs_ref[...]
        t = jnp.maximum(
            jnp.dot(ea_ref[...], w1r[...], preferred_element_type=jnp.float32)
            + b1r[...], 0.0)
        t = jnp.maximum(
            jnp.dot(t, w2r[...], preferred_element_type=jnp.float32)
            + b2r[...], 0.0)
        p = jnp.dot(t, w3r[...], preferred_element_type=jnp.float32)
        hrep = jnp.dot(hs, rr[...], preferred_element_type=jnp.float32)
        o_ref[...] = (
            jnp.dot(hrep * p, cr[...], preferred_element_type=jnp.float32)
            + jnp.dot(hs, bgr[...], preferred_element_type=jnp.float32))

    eg = _E // _G
    return pl.pallas_call(
        body,
        grid=(eg // _EBG,),
        in_specs=[pl.BlockSpec((_EBG, _G * _F), lambda i: (i, 0)),
                  pl.BlockSpec((_EBG, _G * _F), lambda i: (i, 0)),
                  _full_spec(w1), _full_spec(b1), _full_spec(w2),
                  _full_spec(b2), _full_spec(w3), _full_spec(b3),
                  _full_spec(rmat), _full_spec(cmat)],
        out_specs=pl.BlockSpec((_EBG, _G * _F), lambda i: (i, 0)),
        out_shape=jax.ShapeDtypeStruct((eg, _G * _F), jnp.float32),
    )(eag, hsg, w1, b1, w2, b2, w3, b3, rmat, cmat)


def _tc_finalize(agg2g, cnt2g, hg, rootg, biasg):
    def body(a_ref, c_ref, h_ref, rr, br, o_ref):
        a = a_ref[0] + a_ref[1]
        cnt = c_ref[0] + c_ref[1]
        mean = a / jnp.maximum(cnt, 1.0)
        o_ref[...] = jnp.maximum(
            mean
            + jnp.dot(h_ref[...], rr[...], preferred_element_type=jnp.float32)
            + br[...], 0.0)

    ng = _N // _G
    return pl.pallas_call(
        body,
        grid=(1,),
        in_specs=[pl.BlockSpec((_NC, ng, _G * _F), lambda i: (0, 0, 0)),
                  pl.BlockSpec((_NC, ng, _G * _F), lambda i: (0, 0, 0)),
                  pl.BlockSpec((ng, _G * _F), lambda i: (0, 0)),
                  _full_spec(rootg), _full_spec(biasg)],
        out_specs=pl.BlockSpec((ng, _G * _F), lambda i: (0, 0)),
        out_shape=jax.ShapeDtypeStruct((ng, _G * _F), jnp.float32),
    )(agg2g, cnt2g, hg, rootg, biasg)


def _sc_gather(h, src2):
    @functools.partial(
        pl.kernel,
        out_type=jax.ShapeDtypeStruct((_E, _F), jnp.float32),
        mesh=_vmesh(),
        scratch_types=[pltpu.VMEM((_CPW, _CHUNK), jnp.int32),
                       pltpu.VMEM((_EPW, _F), jnp.float32),
                       pltpu.SemaphoreType.DMA,
                       pltpu.SemaphoreType.DMA],
        compiler_params=_sc_params(),
    )
    def k(h_hbm, s_hbm, o_hbm, idx_v, rows_v, sem0, sem1):
        wid = lax.axis_index("s") * _NC + lax.axis_index("c")
        r0 = wid * _CPW
        e0 = wid * _EPW
        pltpu.async_copy(s_hbm.at[pl.ds(r0, _CPW)], idx_v, sem0).wait()

        @pl.loop(0, _CPW)
        def _fire(j):
            pltpu.async_copy(h_hbm.at[idx_v.at[j]],
                             rows_v.at[pl.ds(j * _CHUNK, _CHUNK)], sem1)

        pltpu.make_async_copy(o_hbm.at[pl.ds(e0, _EPW)], rows_v, sem1).wait()
        pltpu.async_copy(rows_v, o_hbm.at[pl.ds(e0, _EPW)], sem0).wait()

    return k(h, src2)


def _sc_scatter(msg, dst2, zeros):
    @functools.partial(
        pl.kernel,
        out_type=jax.ShapeDtypeStruct((_NC, _N, _F), jnp.float32),
        mesh=_vmesh(),
        scratch_types=[pltpu.VMEM((_CPW, _CHUNK), jnp.int32),
                       pltpu.VMEM((_EPW, _F), jnp.float32),
                       pltpu.VMEM_SHARED((_N, _F), jnp.float32),
                       pltpu.SemaphoreType.DMA,
                       pltpu.SemaphoreType.DMA],
        compiler_params=_sc_params(),
    )
    def k(m_hbm, d_hbm, z_hbm, o_hbm, idx_v, msg_v, acc_sh, sem0, sem1):
        cid = lax.axis_index("c")
        sid = lax.axis_index("s")
        wid = sid * _NC + cid
        r0 = wid * _CPW
        e0 = wid * _EPW
        n0 = sid * _NPT
        cp_z = pltpu.async_copy(z_hbm.at[pl.ds(n0, _NPT)],
                                acc_sh.at[pl.ds(n0, _NPT)], sem0)
        cp_i = pltpu.async_copy(d_hbm.at[pl.ds(r0, _CPW)], idx_v, sem1)
        cp_m = pltpu.async_copy(m_hbm.at[pl.ds(e0, _EPW)], msg_v, sem1)
        cp_z.wait()
        cp_i.wait()
        cp_m.wait()
        plsc.subcore_barrier()

        @pl.loop(0, _CPW)
        def _fire(j):
            pltpu.async_copy(msg_v.at[pl.ds(j * _CHUNK, _CHUNK)],
                             acc_sh.at[idx_v.at[j]], sem0, add=True)

        pltpu.make_async_copy(m_hbm.at[pl.ds(e0, _EPW)], msg_v, sem0).wait()
        plsc.subcore_barrier()
        pltpu.sync_copy(acc_sh.at[pl.ds(n0, _NPT)],
                        o_hbm.at[cid, pl.ds(n0, _NPT)])

    return k(msg, dst2, zeros)


def _sc_count(dst2, zeros):
    @functools.partial(
        pl.kernel,
        out_type=jax.ShapeDtypeStruct((_NC, _N, _F), jnp.float32),
        mesh=_vmesh(),
        scratch_types=[pltpu.VMEM((_CPW, _CHUNK), jnp.int32),
                       pltpu.VMEM((_CHUNK, _F), jnp.float32),
                       pltpu.VMEM_SHARED((_N, _F), jnp.float32),
                       pltpu.SemaphoreType.DMA],
        compiler_params=_sc_params(),
    )
    def k(d_hbm, z_hbm, o_hbm, idx_v, ones_v, acc_sh, sem0):
        cid = lax.axis_index("c")
        sid = lax.axis_index("s")
        wid = sid * _NC + cid
        r0 = wid * _CPW
        n0 = sid * _NPT
        cp_z = pltpu.async_copy(z_hbm.at[pl.ds(n0, _NPT)],
                                acc_sh.at[pl.ds(n0, _NPT)], sem0)
        cp_i = pltpu.async_copy(d_hbm.at[pl.ds(r0, _CPW)], idx_v, sem0)

        @pl.loop(0, _CHUNK)
        def _fill(i):
            ones_v[i, :] = jnp.ones((_F,), jnp.float32)

        cp_z.wait()
        cp_i.wait()
        plsc.subcore_barrier()

        @pl.loop(0, _CPW)
        def _fire(j):
            pltpu.sync_copy(ones_v, acc_sh.at[idx_v.at[j]], add=True)

        plsc.subcore_barrier()
        pltpu.sync_copy(acc_sh.at[pl.ds(n0, _NPT)],
                        o_hbm.at[cid, pl.ds(n0, _NPT)])

    return k(dst2, zeros)


def kernel(x, edge_index, edge_attr, params):
    src2 = edge_index[0].reshape(_NROWS, _CHUNK)
    dst2 = edge_index[1].reshape(_NROWS, _CHUNK)
    zeros = jnp.zeros((_N, _F), jnp.float32)
    eye8 = jnp.eye(_G, dtype=jnp.float32)
    kron8 = lambda w: jnp.kron(eye8, w)
    tile8 = lambda b: jnp.tile(b, (_G,)).reshape(1, -1)

    rmat = (jnp.arange(_F * _F)[None, :] // _F
            == jnp.arange(_F)[:, None]).astype(jnp.float32)
    cmat = (jnp.arange(_F * _F)[:, None] % _F
            == jnp.arange(_F)[None, :]).astype(jnp.float32)

    e = params['emb']
    xg = x.reshape(_N // _G, -1)
    hg = _tc_embed(xg, kron8(e['W1']), tile8(e['b1']),
                   kron8(e['W2']), tile8(e['b2']),
                   kron8(e['W3']), tile8(e['b3']))
    cnt2 = _sc_count(dst2, zeros)
    cnt2g = cnt2.reshape(_NC, _N // _G, _G * _F)

    eag = edge_attr.reshape(_E // _G, _G * _F)
    for c in params['convs']:
        hs = _sc_gather(hg.reshape(_N, _F), src2)
        bmat = c['eb3'].reshape(_F, _F)
        msgg = _tc_msg(eag, hs.reshape(_E // _G, _G * _F),
                       kron8(c['eW1']), tile8(c['eb1']),
                       kron8(c['eW2']), tile8(c['eb2']),
                       kron8(c['eW3']), kron8(bmat),
                       kron8(rmat), kron8(cmat))
        agg2 = _sc_scatter(msgg.reshape(_E, _F), dst2, zeros)
        hg = _tc_finalize(agg2.reshape(_NC, _N // _G, _G * _F), cnt2g,
                          hg, kron8(c['root']), tile8(c['bias']))
    return hg.reshape(_N, _F)
